# Initial kernel scaffold; baseline (speedup 1.0000x reference)
import jax
import jax.numpy as jnp
from jax.experimental import pallas as pl


def kernel(x_temporal, num_frames, gcn_W0, gcn_b0, gcn_W1, gcn_b1, gcn_W2, gcn_b2, tin_W, tin_b, Wq0, Wk0, Wv0, Wo0, ff1_0, ff1b_0, ff2_0, ff2b_0, ln1g_0, ln1b_0, ln2g_0, ln2b_0, Wq1, Wk1, Wv1, Wo1, ff1_1, ff1b_1, ff2_1, ff2b_1, ln1g_1, ln1b_1, ln2g_1, ln2b_1, cls_W0, cls_b0, cls_ln0g, cls_ln0b, cls_W1, cls_b1, cls_ln1g, cls_ln1b, cls_W2, cls_b2):
    raise NotImplementedError("write your pallas kernel here")


# speedup vs baseline: 39.5370x; 39.5370x over previous
import functools

import numpy as np
import jax
import jax.numpy as jnp
from jax.experimental import pallas as pl
from jax.experimental.pallas import tpu as pltpu

_N = 96
_K = 5
_GB = 16
_ROWS = _GB * _N
_G = 256
_F = 256
_H = 128
_HT = 256
_NH = 8
_DH = 32
_B = 8
_T = 32


def _band_coeffs() -> np.ndarray:
    src, dst = [], []
    for i in range(_N):
        for j in range(max(0, i - _K), min(_N, i + _K + 1)):
            if i != j:
                src.append(i)
                dst.append(j)
    src = np.asarray(src)
    dst = np.asarray(dst)
    deg = np.zeros(_N, np.float64)
    np.add.at(deg, dst, 1.0)
    deg += 1.0
    dinv = 1.0 / np.sqrt(deg)
    A = np.zeros((_N, _N), np.float64)
    A[dst, src] = dinv[src] * dinv[dst]
    A[np.arange(_N), np.arange(_N)] = dinv * dinv
    C = np.zeros((_N, 2 * _K + 1), np.float64)
    r = np.arange(_N)
    for j, d in enumerate(range(-_K, _K + 1)):
        ok = (r + d >= 0) & (r + d < _N)
        C[ok, j] = A[r[ok], r[ok] + d]
    return np.tile(C, (_GB, 1)).astype(np.float32)


_DOT = functools.partial(jnp.dot, preferred_element_type=jnp.float32,
                         precision=jax.lax.Precision.HIGHEST)


def _spatial_kernel(x_ref, w0_ref, b0_ref, w1_ref, b1_ref, w2_ref, b2_ref,
                    c_ref, ge_ref):
    c = c_ref[:]

    def band(h):
        z = jnp.zeros((8, _H), jnp.float32)
        hp = jnp.concatenate([z, h, z], axis=0)
        acc = None
        for j, d in enumerate(range(-_K, _K + 1)):
            t = c[:, j:j + 1] * jax.lax.slice(hp, (8 + d, 0),
                                              (8 + d + _ROWS, _H))
            acc = t if acc is None else acc + t
        return acc

    h = _DOT(x_ref[:], w0_ref[:])
    h = jnp.maximum(band(h) + b0_ref[:], 0.0)
    h = _DOT(h, w1_ref[:])
    h = jnp.maximum(band(h) + b1_ref[:], 0.0)
    h = _DOT(h, w2_ref[:])
    h = band(h) + b2_ref[:]
    for g in range(_GB):
        seg = h[g * _N:(g + 1) * _N, :]
        ge_ref[g:g + 1, 0:_H] = jnp.mean(seg, axis=0, keepdims=True)
        ge_ref[g:g + 1, _H:2 * _H] = jnp.max(seg, axis=0, keepdims=True)


def _ln(x, g, b):
    m = jnp.mean(x, axis=-1, keepdims=True)
    v = jnp.mean((x - m) ** 2, axis=-1, keepdims=True)
    return (x - m) * jax.lax.rsqrt(v + 1e-5) * g + b


def _temporal_kernel(ge_ref, nf_ref, tinw_ref, tinb_ref,
                     wq0_ref, wk0_ref, wv0_ref, wo0_ref, f10_ref, f1b0_ref,
                     f20_ref, f2b0_ref, g10_ref, b10_ref, g20_ref, b20_ref,
                     wq1_ref, wk1_ref, wv1_ref, wo1_ref, f11_ref, f1b1_ref,
                     f21_ref, f2b1_ref, g11_ref, b11_ref, g21_ref, b21_ref,
                     cw0_ref, cb0_ref, cg0_ref, cbt0_ref,
                     cw1_ref, cb1_ref, cg1_ref, cbt1_ref,
                     cw2_ref, cb2_ref, out_ref, o_scr):
    hT = _DOT(ge_ref[:], tinw_ref[:]) + tinb_ref[:]
    nf = nf_ref[:]
    t_iota = jax.lax.broadcasted_iota(jnp.int32, (_B, _T), 1)
    pad = t_iota >= nf
    scale = jnp.float32(1.0 / np.sqrt(_DH))

    layers = (
        (wq0_ref, wk0_ref, wv0_ref, wo0_ref, f10_ref, f1b0_ref, f20_ref,
         f2b0_ref, g10_ref, b10_ref, g20_ref, b20_ref),
        (wq1_ref, wk1_ref, wv1_ref, wo1_ref, f11_ref, f1b1_ref, f21_ref,
         f2b1_ref, g11_ref, b11_ref, g21_ref, b21_ref),
    )
    for (wq, wk, wv, wo, f1, f1b, f2, f2b, g1, b1, g2, b2) in layers:
        q = _DOT(hT, wq[:])
        k = _DOT(hT, wk[:])
        v = _DOT(hT, wv[:])
        for bi in range(_B):
            r0 = bi * _T
            pad_b = pad[bi:bi + 1, :]
            for hh in range(_NH):
                c0 = hh * _DH
                qb = q[r0:r0 + _T, c0:c0 + _DH]
                kb = k[r0:r0 + _T, c0:c0 + _DH]
                vb = v[r0:r0 + _T, c0:c0 + _DH]
                sc = jax.lax.dot_general(
                    qb, kb, (((1,), (1,)), ((), ())),
                    preferred_element_type=jnp.float32,
                    precision=jax.lax.Precision.HIGHEST) * scale
                sc = jnp.where(pad_b, jnp.float32(-1e9), sc)
                mx = jnp.max(sc, axis=-1, keepdims=True)
                e = jnp.exp(sc - mx)
                at = e / jnp.sum(e, axis=-1, keepdims=True)
                o_scr[r0:r0 + _T, c0:c0 + _DH] = _DOT(at, vb)
        hT = _ln(hT + _DOT(o_scr[:], wo[:]), g1[:], b1[:])
        f = _DOT(jnp.maximum(_DOT(hT, f1[:]) + f1b[:], 0.0), f2[:]) + f2b[:]
        hT = _ln(hT + f, g2[:], b2[:])

    valid = jnp.where(pad, 0.0, 1.0).astype(jnp.float32)
    pooled_rows = []
    for bi in range(_B):
        vb = valid[bi:bi + 1, :]
        pooled_rows.append(_DOT(vb, hT[bi * _T:(bi + 1) * _T, :]))
    pooled = jnp.concatenate(pooled_rows, axis=0) / nf.astype(jnp.float32)

    z = jnp.maximum(_ln(_DOT(pooled, cw0_ref[:]) + cb0_ref[:],
                        cg0_ref[:], cbt0_ref[:]), 0.0)
    z = jnp.maximum(_ln(_DOT(z, cw1_ref[:]) + cb1_ref[:],
                        cg1_ref[:], cbt1_ref[:]), 0.0)
    out_ref[:] = _DOT(z, cw2_ref[:]) + cb2_ref[:]


def kernel(x_temporal, num_frames, gcn_W0, gcn_b0, gcn_W1, gcn_b1, gcn_W2,
           gcn_b2, tin_W, tin_b, Wq0, Wk0, Wv0, Wo0, ff1_0, ff1b_0, ff2_0,
           ff2b_0, ln1g_0, ln1b_0, ln2g_0, ln2b_0, Wq1, Wk1, Wv1, Wo1, ff1_1,
           ff1b_1, ff2_1, ff2b_1, ln1g_1, ln1b_1, ln2g_1, ln2b_1, cls_W0,
           cls_b0, cls_ln0g, cls_ln0b, cls_W1, cls_b1, cls_ln1g, cls_ln1b,
           cls_W2, cls_b2):
    Bb, Tt, Nn, Ff = x_temporal.shape
    xf = x_temporal.reshape(Bb * Tt * Nn, Ff)
    C = jnp.asarray(_band_coeffs())
    r2 = lambda a: a.reshape(1, -1)

    n_chunks = _G // _GB
    ge = pl.pallas_call(
        _spatial_kernel,
        grid=(n_chunks,),
        in_specs=[
            pl.BlockSpec((_ROWS, _F), lambda i: (i, 0)),
            pl.BlockSpec((_F, _H), lambda i: (0, 0)),
            pl.BlockSpec((1, _H), lambda i: (0, 0)),
            pl.BlockSpec((_H, _H), lambda i: (0, 0)),
            pl.BlockSpec((1, _H), lambda i: (0, 0)),
            pl.BlockSpec((_H, _H), lambda i: (0, 0)),
            pl.BlockSpec((1, _H), lambda i: (0, 0)),
            pl.BlockSpec((_ROWS, 2 * _K + 1), lambda i: (0, 0)),
        ],
        out_specs=pl.BlockSpec((_GB, 2 * _H), lambda i: (i, 0)),
        out_shape=jax.ShapeDtypeStruct((_G, 2 * _H), jnp.float32),
        compiler_params=pltpu.CompilerParams(
            dimension_semantics=("parallel",)),
    )(xf, gcn_W0, r2(gcn_b0), gcn_W1, r2(gcn_b1), gcn_W2, r2(gcn_b2), C)

    out = pl.pallas_call(
        _temporal_kernel,
        out_shape=jax.ShapeDtypeStruct((_B, 8), jnp.float32),
        scratch_shapes=[pltpu.VMEM((_B * _T, _HT), jnp.float32)],
    )(ge, num_frames, tin_W, r2(tin_b),
      Wq0, Wk0, Wv0, Wo0, ff1_0, r2(ff1b_0), ff2_0, r2(ff2b_0),
      r2(ln1g_0), r2(ln1b_0), r2(ln2g_0), r2(ln2b_0),
      Wq1, Wk1, Wv1, Wo1, ff1_1, r2(ff1b_1), ff2_1, r2(ff2b_1),
      r2(ln1g_1), r2(ln1b_1), r2(ln2g_1), r2(ln2b_1),
      cls_W0, r2(cls_b0), r2(cls_ln0g), r2(cls_ln0b),
      cls_W1, r2(cls_b1), r2(cls_ln1g), r2(cls_ln1b),
      cls_W2, r2(cls_b2))
    return out


# speedup vs baseline: 61.2256x; 1.5486x over previous
import functools

import numpy as np
import jax
import jax.numpy as jnp
from jax.experimental import pallas as pl
from jax.experimental.pallas import tpu as pltpu

_N = 96
_K = 5
_GB = 16
_ROWS = _GB * _N
_G = 256
_F = 256
_H = 128
_HT = 256
_NH = 8
_DH = 32
_B = 8
_T = 32


def _band_coeffs() -> np.ndarray:
    src, dst = [], []
    for i in range(_N):
        for j in range(max(0, i - _K), min(_N, i + _K + 1)):
            if i != j:
                src.append(i)
                dst.append(j)
    src = np.asarray(src)
    dst = np.asarray(dst)
    deg = np.zeros(_N, np.float64)
    np.add.at(deg, dst, 1.0)
    deg += 1.0
    dinv = 1.0 / np.sqrt(deg)
    A = np.zeros((_N, _N), np.float64)
    A[dst, src] = dinv[src] * dinv[dst]
    A[np.arange(_N), np.arange(_N)] = dinv * dinv
    C = np.zeros((_N, 2 * _K + 1), np.float64)
    r = np.arange(_N)
    for j, d in enumerate(range(-_K, _K + 1)):
        ok = (r + d >= 0) & (r + d < _N)
        C[ok, j] = A[r[ok], r[ok] + d]
    return np.tile(C, (_GB, 1)).astype(np.float32)


_DOT = functools.partial(jnp.dot, preferred_element_type=jnp.float32,
                         precision=jax.lax.Precision.DEFAULT)


def _spatial_kernel(x_ref, w0_ref, b0_ref, w1_ref, b1_ref, w2_ref, b2_ref,
                    c_ref, ge_ref):
    c = c_ref[:]

    def band(h):
        z = jnp.zeros((8, _H), jnp.float32)
        hp = jnp.concatenate([z, h, z], axis=0)
        acc = None
        for j, d in enumerate(range(-_K, _K + 1)):
            t = c[:, j:j + 1] * jax.lax.slice(hp, (8 + d, 0),
                                              (8 + d + _ROWS, _H))
            acc = t if acc is None else acc + t
        return acc

    h = _DOT(x_ref[:], w0_ref[:])
    h = jnp.maximum(band(h) + b0_ref[:], 0.0)
    h = _DOT(h, w1_ref[:])
    h = jnp.maximum(band(h) + b1_ref[:], 0.0)
    h = _DOT(h, w2_ref[:])
    h = band(h) + b2_ref[:]
    for g in range(_GB):
        seg = h[g * _N:(g + 1) * _N, :]
        ge_ref[g:g + 1, 0:_H] = jnp.mean(seg, axis=0, keepdims=True)
        ge_ref[g:g + 1, _H:2 * _H] = jnp.max(seg, axis=0, keepdims=True)


def _ln(x, g, b):
    m = jnp.mean(x, axis=-1, keepdims=True)
    v = jnp.mean((x - m) ** 2, axis=-1, keepdims=True)
    return (x - m) * jax.lax.rsqrt(v + 1e-5) * g + b


def _temporal_kernel(ge_ref, nf_ref, tinw_ref, tinb_ref,
                     wq0_ref, wk0_ref, wv0_ref, wo0_ref, f10_ref, f1b0_ref,
                     f20_ref, f2b0_ref, g10_ref, b10_ref, g20_ref, b20_ref,
                     wq1_ref, wk1_ref, wv1_ref, wo1_ref, f11_ref, f1b1_ref,
                     f21_ref, f2b1_ref, g11_ref, b11_ref, g21_ref, b21_ref,
                     cw0_ref, cb0_ref, cg0_ref, cbt0_ref,
                     cw1_ref, cb1_ref, cg1_ref, cbt1_ref,
                     cw2_ref, cb2_ref, out_ref, o_scr):
    hT = _DOT(ge_ref[:], tinw_ref[:]) + tinb_ref[:]
    nf = nf_ref[:]
    t_iota = jax.lax.broadcasted_iota(jnp.int32, (_B, _T), 1)
    pad = t_iota >= nf
    scale = jnp.float32(1.0 / np.sqrt(_DH))

    layers = (
        (wq0_ref, wk0_ref, wv0_ref, wo0_ref, f10_ref, f1b0_ref, f20_ref,
         f2b0_ref, g10_ref, b10_ref, g20_ref, b20_ref),
        (wq1_ref, wk1_ref, wv1_ref, wo1_ref, f11_ref, f1b1_ref, f21_ref,
         f2b1_ref, g11_ref, b11_ref, g21_ref, b21_ref),
    )
    for (wq, wk, wv, wo, f1, f1b, f2, f2b, g1, b1, g2, b2) in layers:
        q = _DOT(hT, wq[:])
        k = _DOT(hT, wk[:])
        v = _DOT(hT, wv[:])
        for bi in range(_B):
            r0 = bi * _T
            pad_b = pad[bi:bi + 1, :]
            for hh in range(_NH):
                c0 = hh * _DH
                qb = q[r0:r0 + _T, c0:c0 + _DH]
                kb = k[r0:r0 + _T, c0:c0 + _DH]
                vb = v[r0:r0 + _T, c0:c0 + _DH]
                sc = jax.lax.dot_general(
                    qb, kb, (((1,), (1,)), ((), ())),
                    preferred_element_type=jnp.float32,
                    precision=jax.lax.Precision.DEFAULT) * scale
                sc = jnp.where(pad_b, jnp.float32(-1e9), sc)
                mx = jnp.max(sc, axis=-1, keepdims=True)
                e = jnp.exp(sc - mx)
                at = e / jnp.sum(e, axis=-1, keepdims=True)
                o_scr[r0:r0 + _T, c0:c0 + _DH] = _DOT(at, vb)
        hT = _ln(hT + _DOT(o_scr[:], wo[:]), g1[:], b1[:])
        f = _DOT(jnp.maximum(_DOT(hT, f1[:]) + f1b[:], 0.0), f2[:]) + f2b[:]
        hT = _ln(hT + f, g2[:], b2[:])

    valid = jnp.where(pad, 0.0, 1.0).astype(jnp.float32)
    pooled_rows = []
    for bi in range(_B):
        vb = valid[bi:bi + 1, :]
        pooled_rows.append(_DOT(vb, hT[bi * _T:(bi + 1) * _T, :]))
    pooled = jnp.concatenate(pooled_rows, axis=0) / nf.astype(jnp.float32)

    z = jnp.maximum(_ln(_DOT(pooled, cw0_ref[:]) + cb0_ref[:],
                        cg0_ref[:], cbt0_ref[:]), 0.0)
    z = jnp.maximum(_ln(_DOT(z, cw1_ref[:]) + cb1_ref[:],
                        cg1_ref[:], cbt1_ref[:]), 0.0)
    out_ref[:] = _DOT(z, cw2_ref[:]) + cb2_ref[:]


def kernel(x_temporal, num_frames, gcn_W0, gcn_b0, gcn_W1, gcn_b1, gcn_W2,
           gcn_b2, tin_W, tin_b, Wq0, Wk0, Wv0, Wo0, ff1_0, ff1b_0, ff2_0,
           ff2b_0, ln1g_0, ln1b_0, ln2g_0, ln2b_0, Wq1, Wk1, Wv1, Wo1, ff1_1,
           ff1b_1, ff2_1, ff2b_1, ln1g_1, ln1b_1, ln2g_1, ln2b_1, cls_W0,
           cls_b0, cls_ln0g, cls_ln0b, cls_W1, cls_b1, cls_ln1g, cls_ln1b,
           cls_W2, cls_b2):
    Bb, Tt, Nn, Ff = x_temporal.shape
    xf = x_temporal.reshape(Bb * Tt * Nn, Ff)
    C = jnp.asarray(_band_coeffs())
    r2 = lambda a: a.reshape(1, -1)

    n_chunks = _G // _GB
    ge = pl.pallas_call(
        _spatial_kernel,
        grid=(n_chunks,),
        in_specs=[
            pl.BlockSpec((_ROWS, _F), lambda i: (i, 0)),
            pl.BlockSpec((_F, _H), lambda i: (0, 0)),
            pl.BlockSpec((1, _H), lambda i: (0, 0)),
            pl.BlockSpec((_H, _H), lambda i: (0, 0)),
            pl.BlockSpec((1, _H), lambda i: (0, 0)),
            pl.BlockSpec((_H, _H), lambda i: (0, 0)),
            pl.BlockSpec((1, _H), lambda i: (0, 0)),
            pl.BlockSpec((_ROWS, 2 * _K + 1), lambda i: (0, 0)),
        ],
        out_specs=pl.BlockSpec((_GB, 2 * _H), lambda i: (i, 0)),
        out_shape=jax.ShapeDtypeStruct((_G, 2 * _H), jnp.float32),
        compiler_params=pltpu.CompilerParams(
            dimension_semantics=("parallel",)),
    )(xf, gcn_W0, r2(gcn_b0), gcn_W1, r2(gcn_b1), gcn_W2, r2(gcn_b2), C)

    out = pl.pallas_call(
        _temporal_kernel,
        out_shape=jax.ShapeDtypeStruct((_B, 8), jnp.float32),
        scratch_shapes=[pltpu.VMEM((_B * _T, _HT), jnp.float32)],
    )(ge, num_frames, tin_W, r2(tin_b),
      Wq0, Wk0, Wv0, Wo0, ff1_0, r2(ff1b_0), ff2_0, r2(ff2b_0),
      r2(ln1g_0), r2(ln1b_0), r2(ln2g_0), r2(ln2b_0),
      Wq1, Wk1, Wv1, Wo1, ff1_1, r2(ff1b_1), ff2_1, r2(ff2b_1),
      r2(ln1g_1), r2(ln1b_1), r2(ln2g_1), r2(ln2b_1),
      cls_W0, r2(cls_b0), r2(cls_ln0g), r2(cls_ln0b),
      cls_W1, r2(cls_b1), r2(cls_ln1g), r2(cls_ln1b),
      cls_W2, r2(cls_b2))
    return out


# speedup vs baseline: 89.7793x; 1.4664x over previous
import functools

import numpy as np
import jax
import jax.numpy as jnp
from jax.experimental import pallas as pl
from jax.experimental.pallas import tpu as pltpu

_N = 96
_K = 5
_GB = 16
_ROWS = _GB * _N
_G = 256
_F = 256
_H = 128
_HT = 256
_NH = 8
_DH = 32
_B = 8
_T = 32


def _band_coeffs() -> np.ndarray:
    src, dst = [], []
    for i in range(_N):
        for j in range(max(0, i - _K), min(_N, i + _K + 1)):
            if i != j:
                src.append(i)
                dst.append(j)
    src = np.asarray(src)
    dst = np.asarray(dst)
    deg = np.zeros(_N, np.float64)
    np.add.at(deg, dst, 1.0)
    deg += 1.0
    dinv = 1.0 / np.sqrt(deg)
    A = np.zeros((_N, _N), np.float64)
    A[dst, src] = dinv[src] * dinv[dst]
    A[np.arange(_N), np.arange(_N)] = dinv * dinv
    C = np.zeros((_N, 2 * _K + 1), np.float64)
    r = np.arange(_N)
    for j, d in enumerate(range(-_K, _K + 1)):
        ok = (r + d >= 0) & (r + d < _N)
        C[ok, j] = A[r[ok], r[ok] + d]
    return np.tile(C, (_GB, 1)).astype(np.float32)


_DOT = functools.partial(jnp.dot, preferred_element_type=jnp.float32,
                         precision=jax.lax.Precision.DEFAULT)


def _spatial_kernel(x_ref, w0_ref, b0_ref, w1_ref, b1_ref, w2_ref, b2_ref,
                    c_ref, ge_ref):
    c = c_ref[:]

    def band(h):
        z = jnp.zeros((8, _H), jnp.float32)
        hp = jnp.concatenate([z, h, z], axis=0)
        acc = None
        for j, d in enumerate(range(-_K, _K + 1)):
            t = c[:, j:j + 1] * jax.lax.slice(hp, (8 + d, 0),
                                              (8 + d + _ROWS, _H))
            acc = t if acc is None else acc + t
        return acc

    h = _DOT(x_ref[:], w0_ref[:])
    h = jnp.maximum(band(h) + b0_ref[:], 0.0)
    h = _DOT(h, w1_ref[:])
    h = jnp.maximum(band(h) + b1_ref[:], 0.0)
    h = _DOT(h, w2_ref[:])
    h = band(h) + b2_ref[:]
    for g in range(_GB):
        seg = h[g * _N:(g + 1) * _N, :]
        ge_ref[g:g + 1, 0:_H] = jnp.mean(seg, axis=0, keepdims=True)
        ge_ref[g:g + 1, _H:2 * _H] = jnp.max(seg, axis=0, keepdims=True)


def _ln(x, g, b):
    m = jnp.mean(x, axis=-1, keepdims=True)
    v = jnp.mean((x - m) ** 2, axis=-1, keepdims=True)
    return (x - m) * jax.lax.rsqrt(v + 1e-5) * g + b


def _temporal_kernel(ge_ref, nf_ref, tinw_ref, tinb_ref,
                     wq0_ref, wk0_ref, wv0_ref, wo0_ref, f10_ref, f1b0_ref,
                     f20_ref, f2b0_ref, g10_ref, b10_ref, g20_ref, b20_ref,
                     wq1_ref, wk1_ref, wv1_ref, wo1_ref, f11_ref, f1b1_ref,
                     f21_ref, f2b1_ref, g11_ref, b11_ref, g21_ref, b21_ref,
                     cw0_ref, cb0_ref, cg0_ref, cbt0_ref,
                     cw1_ref, cb1_ref, cg1_ref, cbt1_ref,
                     cw2_ref, cb2_ref, out_ref, o_scr):
    del o_scr
    hT = _DOT(ge_ref[:], tinw_ref[:]) + tinb_ref[:]
    nf = nf_ref[:]
    t_iota = jax.lax.broadcasted_iota(jnp.int32, (_B, _T), 1)
    pad = t_iota >= nf
    scale = jnp.float32(1.0 / np.sqrt(_DH))
    BT = _B * _T

    r_blk = jax.lax.broadcasted_iota(jnp.int32, (BT, BT), 0) // _DH
    c_blk = jax.lax.broadcasted_iota(jnp.int32, (BT, BT), 1) // _DH
    M = (r_blk == c_blk).astype(jnp.float32)

    mrows = []
    for bi in range(_B):
        row = jnp.where(pad[bi:bi + 1, :], jnp.float32(-1e9), 0.0)
        rowt = jnp.concatenate([row] * _NH, axis=1)
        mrows.append(jnp.broadcast_to(rowt, (_T, BT)))
    addend = jnp.concatenate(mrows, axis=0)

    layers = (
        (wq0_ref, wk0_ref, wv0_ref, wo0_ref, f10_ref, f1b0_ref, f20_ref,
         f2b0_ref, g10_ref, b10_ref, g20_ref, b20_ref),
        (wq1_ref, wk1_ref, wv1_ref, wo1_ref, f11_ref, f1b1_ref, f21_ref,
         f2b1_ref, g11_ref, b11_ref, g21_ref, b21_ref),
    )
    for (wq, wk, wv, wo, f1, f1b, f2, f2b, g1, b1, g2, b2) in layers:
        q = _DOT(hT, wq[:])
        k = _DOT(hT, wk[:])
        v = _DOT(hT, wv[:])
        kT = k.T
        srows = []
        for bi in range(_B):
            kTb = kT[:, bi * _T:(bi + 1) * _T]
            Kp = jnp.concatenate([kTb] * _NH, axis=1) * M
            srows.append(_DOT(q[bi * _T:(bi + 1) * _T, :], Kp))
        S2 = jnp.concatenate(srows, axis=0) * scale + addend
        mx = jnp.max(S2)
        e = jnp.exp(S2 - mx)
        denom = _DOT(e, M)
        A2 = e / denom
        orows = []
        for bi in range(_B):
            vb = v[bi * _T:(bi + 1) * _T, :]
            Vp = jnp.concatenate([vb] * _NH, axis=0) * M
            orows.append(_DOT(A2[bi * _T:(bi + 1) * _T, :], Vp))
        O = jnp.concatenate(orows, axis=0)
        hT = _ln(hT + _DOT(O, wo[:]), g1[:], b1[:])
        f = _DOT(jnp.maximum(_DOT(hT, f1[:]) + f1b[:], 0.0), f2[:]) + f2b[:]
        hT = _ln(hT + f, g2[:], b2[:])

    valid = jnp.where(pad, 0.0, 1.0).astype(jnp.float32)
    pooled_rows = []
    for bi in range(_B):
        vb = valid[bi:bi + 1, :]
        pooled_rows.append(_DOT(vb, hT[bi * _T:(bi + 1) * _T, :]))
    pooled = jnp.concatenate(pooled_rows, axis=0) / nf.astype(jnp.float32)

    z = jnp.maximum(_ln(_DOT(pooled, cw0_ref[:]) + cb0_ref[:],
                        cg0_ref[:], cbt0_ref[:]), 0.0)
    z = jnp.maximum(_ln(_DOT(z, cw1_ref[:]) + cb1_ref[:],
                        cg1_ref[:], cbt1_ref[:]), 0.0)
    out_ref[:] = _DOT(z, cw2_ref[:]) + cb2_ref[:]


def kernel(x_temporal, num_frames, gcn_W0, gcn_b0, gcn_W1, gcn_b1, gcn_W2,
           gcn_b2, tin_W, tin_b, Wq0, Wk0, Wv0, Wo0, ff1_0, ff1b_0, ff2_0,
           ff2b_0, ln1g_0, ln1b_0, ln2g_0, ln2b_0, Wq1, Wk1, Wv1, Wo1, ff1_1,
           ff1b_1, ff2_1, ff2b_1, ln1g_1, ln1b_1, ln2g_1, ln2b_1, cls_W0,
           cls_b0, cls_ln0g, cls_ln0b, cls_W1, cls_b1, cls_ln1g, cls_ln1b,
           cls_W2, cls_b2):
    Bb, Tt, Nn, Ff = x_temporal.shape
    xf = x_temporal.reshape(Bb * Tt * Nn, Ff)
    C = jnp.asarray(_band_coeffs())
    r2 = lambda a: a.reshape(1, -1)

    n_chunks = _G // _GB
    ge = pl.pallas_call(
        _spatial_kernel,
        grid=(n_chunks,),
        in_specs=[
            pl.BlockSpec((_ROWS, _F), lambda i: (i, 0)),
            pl.BlockSpec((_F, _H), lambda i: (0, 0)),
            pl.BlockSpec((1, _H), lambda i: (0, 0)),
            pl.BlockSpec((_H, _H), lambda i: (0, 0)),
            pl.BlockSpec((1, _H), lambda i: (0, 0)),
            pl.BlockSpec((_H, _H), lambda i: (0, 0)),
            pl.BlockSpec((1, _H), lambda i: (0, 0)),
            pl.BlockSpec((_ROWS, 2 * _K + 1), lambda i: (0, 0)),
        ],
        out_specs=pl.BlockSpec((_GB, 2 * _H), lambda i: (i, 0)),
        out_shape=jax.ShapeDtypeStruct((_G, 2 * _H), jnp.float32),
        compiler_params=pltpu.CompilerParams(
            dimension_semantics=("parallel",)),
    )(xf, gcn_W0, r2(gcn_b0), gcn_W1, r2(gcn_b1), gcn_W2, r2(gcn_b2), C)

    out = pl.pallas_call(
        _temporal_kernel,
        out_shape=jax.ShapeDtypeStruct((_B, 8), jnp.float32),
        scratch_shapes=[pltpu.VMEM((_B * _T, _HT), jnp.float32)],
    )(ge, num_frames, tin_W, r2(tin_b),
      Wq0, Wk0, Wv0, Wo0, ff1_0, r2(ff1b_0), ff2_0, r2(ff2b_0),
      r2(ln1g_0), r2(ln1b_0), r2(ln2g_0), r2(ln2b_0),
      Wq1, Wk1, Wv1, Wo1, ff1_1, r2(ff1b_1), ff2_1, r2(ff2b_1),
      r2(ln1g_1), r2(ln1b_1), r2(ln2g_1), r2(ln2b_1),
      cls_W0, r2(cls_b0), r2(cls_ln0g), r2(cls_ln0b),
      cls_W1, r2(cls_b1), r2(cls_ln1g), r2(cls_ln1b),
      cls_W2, r2(cls_b2))
    return out


# speedup vs baseline: 203.3495x; 2.2650x over previous
import functools

import numpy as np
import jax
import jax.numpy as jnp
from jax.experimental import pallas as pl
from jax.experimental.pallas import tpu as pltpu

_N = 96
_K = 5
_GB = 16
_ROWS = _GB * _N
_G = 256
_F = 256
_H = 128
_HT = 256
_NH = 8
_DH = 32
_B = 8
_T = 32


def _band_adjacency() -> np.ndarray:
    src, dst = [], []
    for i in range(_N):
        for j in range(max(0, i - _K), min(_N, i + _K + 1)):
            if i != j:
                src.append(i)
                dst.append(j)
    src = np.asarray(src)
    dst = np.asarray(dst)
    deg = np.zeros(_N, np.float64)
    np.add.at(deg, dst, 1.0)
    deg += 1.0
    dinv = 1.0 / np.sqrt(deg)
    A = np.zeros((_N, _N), np.float64)
    A[dst, src] = dinv[src] * dinv[dst]
    A[np.arange(_N), np.arange(_N)] = dinv * dinv
    return A.astype(np.float32)


_DOT = functools.partial(jnp.dot, preferred_element_type=jnp.float32,
                         precision=jax.lax.Precision.DEFAULT)


def _spatial_kernel(x_ref, w0_ref, b0_ref, w1_ref, b1_ref, w2_ref, b2_ref,
                    a_ref, ge_ref):
    ahat = a_ref[:]

    def band(h):
        htile = jnp.concatenate(
            [h[g * _N:(g + 1) * _N, :] for g in range(_GB)], axis=1)
        otile = _DOT(ahat, htile)
        return jnp.concatenate(
            [otile[:, g * _H:(g + 1) * _H] for g in range(_GB)], axis=0)

    h = _DOT(x_ref[:], w0_ref[:])
    h = jnp.maximum(band(h) + b0_ref[:], 0.0)
    h = _DOT(h, w1_ref[:])
    h = jnp.maximum(band(h) + b1_ref[:], 0.0)
    h = _DOT(h, w2_ref[:])
    h = band(h) + b2_ref[:]
    for g in range(_GB):
        seg = h[g * _N:(g + 1) * _N, :]
        ge_ref[g:g + 1, 0:_H] = jnp.mean(seg, axis=0, keepdims=True)
        ge_ref[g:g + 1, _H:2 * _H] = jnp.max(seg, axis=0, keepdims=True)


def _ln(x, g, b):
    m = jnp.mean(x, axis=-1, keepdims=True)
    v = jnp.mean((x - m) ** 2, axis=-1, keepdims=True)
    return (x - m) * jax.lax.rsqrt(v + 1e-5) * g + b


def _temporal_kernel(ge_ref, nf_ref, tinw_ref, tinb_ref,
                     wq0_ref, wk0_ref, wv0_ref, wo0_ref, f10_ref, f1b0_ref,
                     f20_ref, f2b0_ref, g10_ref, b10_ref, g20_ref, b20_ref,
                     wq1_ref, wk1_ref, wv1_ref, wo1_ref, f11_ref, f1b1_ref,
                     f21_ref, f2b1_ref, g11_ref, b11_ref, g21_ref, b21_ref,
                     cw0_ref, cb0_ref, cg0_ref, cbt0_ref,
                     cw1_ref, cb1_ref, cg1_ref, cbt1_ref,
                     cw2_ref, cb2_ref, out_ref, o_scr):
    del o_scr
    hT = _DOT(ge_ref[:], tinw_ref[:]) + tinb_ref[:]
    nf = nf_ref[:]
    t_iota = jax.lax.broadcasted_iota(jnp.int32, (_B, _T), 1)
    pad = t_iota >= nf
    scale = jnp.float32(1.0 / np.sqrt(_DH))
    BT = _B * _T

    r_blk = jax.lax.broadcasted_iota(jnp.int32, (BT, BT), 0) // _DH
    c_blk = jax.lax.broadcasted_iota(jnp.int32, (BT, BT), 1) // _DH
    M = (r_blk == c_blk).astype(jnp.float32)

    mrows = []
    for bi in range(_B):
        row = jnp.where(pad[bi:bi + 1, :], jnp.float32(-1e9), 0.0)
        rowt = jnp.concatenate([row] * _NH, axis=1)
        mrows.append(jnp.broadcast_to(rowt, (_T, BT)))
    addend = jnp.concatenate(mrows, axis=0)

    layers = (
        (wq0_ref, wk0_ref, wv0_ref, wo0_ref, f10_ref, f1b0_ref, f20_ref,
         f2b0_ref, g10_ref, b10_ref, g20_ref, b20_ref),
        (wq1_ref, wk1_ref, wv1_ref, wo1_ref, f11_ref, f1b1_ref, f21_ref,
         f2b1_ref, g11_ref, b11_ref, g21_ref, b21_ref),
    )
    for (wq, wk, wv, wo, f1, f1b, f2, f2b, g1, b1, g2, b2) in layers:
        q = _DOT(hT, wq[:])
        k = _DOT(hT, wk[:])
        v = _DOT(hT, wv[:])
        kT = k.T
        srows = []
        for bi in range(_B):
            kTb = kT[:, bi * _T:(bi + 1) * _T]
            Kp = jnp.concatenate([kTb] * _NH, axis=1) * M
            srows.append(_DOT(q[bi * _T:(bi + 1) * _T, :], Kp))
        S2 = jnp.concatenate(srows, axis=0) * scale + addend
        mx = jnp.max(S2)
        e = jnp.exp(S2 - mx)
        denom = _DOT(e, M)
        A2 = e / denom
        orows = []
        for bi in range(_B):
            vb = v[bi * _T:(bi + 1) * _T, :]
            Vp = jnp.concatenate([vb] * _NH, axis=0) * M
            orows.append(_DOT(A2[bi * _T:(bi + 1) * _T, :], Vp))
        O = jnp.concatenate(orows, axis=0)
        hT = _ln(hT + _DOT(O, wo[:]), g1[:], b1[:])
        f = _DOT(jnp.maximum(_DOT(hT, f1[:]) + f1b[:], 0.0), f2[:]) + f2b[:]
        hT = _ln(hT + f, g2[:], b2[:])

    valid = jnp.where(pad, 0.0, 1.0).astype(jnp.float32)
    pooled_rows = []
    for bi in range(_B):
        vb = valid[bi:bi + 1, :]
        pooled_rows.append(_DOT(vb, hT[bi * _T:(bi + 1) * _T, :]))
    pooled = jnp.concatenate(pooled_rows, axis=0) / nf.astype(jnp.float32)

    z = jnp.maximum(_ln(_DOT(pooled, cw0_ref[:]) + cb0_ref[:],
                        cg0_ref[:], cbt0_ref[:]), 0.0)
    z = jnp.maximum(_ln(_DOT(z, cw1_ref[:]) + cb1_ref[:],
                        cg1_ref[:], cbt1_ref[:]), 0.0)
    out_ref[:] = _DOT(z, cw2_ref[:]) + cb2_ref[:]


def kernel(x_temporal, num_frames, gcn_W0, gcn_b0, gcn_W1, gcn_b1, gcn_W2,
           gcn_b2, tin_W, tin_b, Wq0, Wk0, Wv0, Wo0, ff1_0, ff1b_0, ff2_0,
           ff2b_0, ln1g_0, ln1b_0, ln2g_0, ln2b_0, Wq1, Wk1, Wv1, Wo1, ff1_1,
           ff1b_1, ff2_1, ff2b_1, ln1g_1, ln1b_1, ln2g_1, ln2b_1, cls_W0,
           cls_b0, cls_ln0g, cls_ln0b, cls_W1, cls_b1, cls_ln1g, cls_ln1b,
           cls_W2, cls_b2):
    Bb, Tt, Nn, Ff = x_temporal.shape
    xf = x_temporal.reshape(Bb * Tt * Nn, Ff)
    A = jnp.asarray(_band_adjacency())
    r2 = lambda a: a.reshape(1, -1)

    n_chunks = _G // _GB
    ge = pl.pallas_call(
        _spatial_kernel,
        grid=(n_chunks,),
        in_specs=[
            pl.BlockSpec((_ROWS, _F), lambda i: (i, 0)),
            pl.BlockSpec((_F, _H), lambda i: (0, 0)),
            pl.BlockSpec((1, _H), lambda i: (0, 0)),
            pl.BlockSpec((_H, _H), lambda i: (0, 0)),
            pl.BlockSpec((1, _H), lambda i: (0, 0)),
            pl.BlockSpec((_H, _H), lambda i: (0, 0)),
            pl.BlockSpec((1, _H), lambda i: (0, 0)),
            pl.BlockSpec((_N, _N), lambda i: (0, 0)),
        ],
        out_specs=pl.BlockSpec((_GB, 2 * _H), lambda i: (i, 0)),
        out_shape=jax.ShapeDtypeStruct((_G, 2 * _H), jnp.float32),
        compiler_params=pltpu.CompilerParams(
            dimension_semantics=("parallel",)),
    )(xf, gcn_W0, r2(gcn_b0), gcn_W1, r2(gcn_b1), gcn_W2, r2(gcn_b2), A)

    out = pl.pallas_call(
        _temporal_kernel,
        out_shape=jax.ShapeDtypeStruct((_B, 8), jnp.float32),
        scratch_shapes=[pltpu.VMEM((_B * _T, _HT), jnp.float32)],
    )(ge, num_frames, tin_W, r2(tin_b),
      Wq0, Wk0, Wv0, Wo0, ff1_0, r2(ff1b_0), ff2_0, r2(ff2b_0),
      r2(ln1g_0), r2(ln1b_0), r2(ln2g_0), r2(ln2b_0),
      Wq1, Wk1, Wv1, Wo1, ff1_1, r2(ff1b_1), ff2_1, r2(ff2b_1),
      r2(ln1g_1), r2(ln1b_1), r2(ln2g_1), r2(ln2b_1),
      cls_W0, r2(cls_b0), r2(cls_ln0g), r2(cls_ln0b),
      cls_W1, r2(cls_b1), r2(cls_ln1g), r2(cls_ln1b),
      cls_W2, r2(cls_b2))
    return out


# speedup vs baseline: 203.9946x; 1.0032x over previous
import functools

import numpy as np
import jax
import jax.numpy as jnp
from jax.experimental import pallas as pl
from jax.experimental.pallas import tpu as pltpu

_N = 96
_K = 5
_GB = 16
_ROWS = _GB * _N
_G = 256
_F = 256
_H = 128
_HT = 256
_NH = 8
_DH = 32
_B = 8
_T = 32


def _band_adjacency() -> np.ndarray:
    src, dst = [], []
    for i in range(_N):
        for j in range(max(0, i - _K), min(_N, i + _K + 1)):
            if i != j:
                src.append(i)
                dst.append(j)
    src = np.asarray(src)
    dst = np.asarray(dst)
    deg = np.zeros(_N, np.float64)
    np.add.at(deg, dst, 1.0)
    deg += 1.0
    dinv = 1.0 / np.sqrt(deg)
    A = np.zeros((_N, _N), np.float64)
    A[dst, src] = dinv[src] * dinv[dst]
    A[np.arange(_N), np.arange(_N)] = dinv * dinv
    return A.astype(np.float32)


_DOT = functools.partial(jnp.dot, preferred_element_type=jnp.float32,
                         precision=jax.lax.Precision.DEFAULT)


def _spatial_kernel(x_ref, w0_ref, b0_ref, w1_ref, b1_ref, w2_ref, b2_ref,
                    a_ref, ge_ref):
    ahat = a_ref[:]

    def band(h):
        htile = jnp.concatenate(
            [h[g * _N:(g + 1) * _N, :] for g in range(_GB)], axis=1)
        otile = _DOT(ahat, htile)
        return jnp.concatenate(
            [otile[:, g * _H:(g + 1) * _H] for g in range(_GB)], axis=0)

    h = _DOT(x_ref[:], w0_ref[:])
    h = jnp.maximum(band(h) + b0_ref[:], 0.0)
    h = _DOT(h, w1_ref[:])
    h = jnp.maximum(band(h) + b1_ref[:], 0.0)
    h = _DOT(h, w2_ref[:])
    h = band(h) + b2_ref[:]
    for g in range(_GB):
        seg = h[g * _N:(g + 1) * _N, :]
        ge_ref[g:g + 1, 0:_H] = jnp.mean(seg, axis=0, keepdims=True)
        ge_ref[g:g + 1, _H:2 * _H] = jnp.max(seg, axis=0, keepdims=True)


def _ln(x, g, b):
    m = jnp.mean(x, axis=-1, keepdims=True)
    v = jnp.mean((x - m) ** 2, axis=-1, keepdims=True)
    return (x - m) * jax.lax.rsqrt(v + 1e-5) * g + b


def _temporal_kernel(ge_ref, nf_ref, tinw_ref, tinb_ref,
                     wq0_ref, wk0_ref, wv0_ref, wo0_ref, f10_ref, f1b0_ref,
                     f20_ref, f2b0_ref, g10_ref, b10_ref, g20_ref, b20_ref,
                     wq1_ref, wk1_ref, wv1_ref, wo1_ref, f11_ref, f1b1_ref,
                     f21_ref, f2b1_ref, g11_ref, b11_ref, g21_ref, b21_ref,
                     cw0_ref, cb0_ref, cg0_ref, cbt0_ref,
                     cw1_ref, cb1_ref, cg1_ref, cbt1_ref,
                     cw2_ref, cb2_ref, out_ref, o_scr):
    del o_scr
    hT = _DOT(ge_ref[:], tinw_ref[:]) + tinb_ref[:]
    nf = nf_ref[:]
    t_iota = jax.lax.broadcasted_iota(jnp.int32, (_B, _T), 1)
    pad = t_iota >= nf
    scale = jnp.float32(1.0 / np.sqrt(_DH))
    BT = _B * _T

    r_blk = jax.lax.broadcasted_iota(jnp.int32, (BT, BT), 0) // _DH
    c_blk = jax.lax.broadcasted_iota(jnp.int32, (BT, BT), 1) // _DH
    M = (r_blk == c_blk).astype(jnp.float32)

    mrows = []
    for bi in range(_B):
        row = jnp.where(pad[bi:bi + 1, :], jnp.float32(-1e9), 0.0)
        rowt = jnp.concatenate([row] * _NH, axis=1)
        mrows.append(jnp.broadcast_to(rowt, (_T, BT)))
    addend = jnp.concatenate(mrows, axis=0)

    layers = (
        (wq0_ref, wk0_ref, wv0_ref, wo0_ref, f10_ref, f1b0_ref, f20_ref,
         f2b0_ref, g10_ref, b10_ref, g20_ref, b20_ref),
        (wq1_ref, wk1_ref, wv1_ref, wo1_ref, f11_ref, f1b1_ref, f21_ref,
         f2b1_ref, g11_ref, b11_ref, g21_ref, b21_ref),
    )
    for (wq, wk, wv, wo, f1, f1b, f2, f2b, g1, b1, g2, b2) in layers:
        q = _DOT(hT, wq[:])
        k = _DOT(hT, wk[:])
        v = _DOT(hT, wv[:])
        kT = k.T
        srows = []
        for bi in range(_B):
            kTb = kT[:, bi * _T:(bi + 1) * _T]
            Kp = jnp.concatenate([kTb] * _NH, axis=1) * M
            srows.append(_DOT(q[bi * _T:(bi + 1) * _T, :], Kp))
        S2 = jnp.concatenate(srows, axis=0) * scale + addend
        mx = jnp.max(S2)
        e = jnp.exp(S2 - mx)
        denom = _DOT(e, M)
        A2 = e / denom
        orows = []
        for bi in range(_B):
            vb = v[bi * _T:(bi + 1) * _T, :]
            Vp = jnp.concatenate([vb] * _NH, axis=0) * M
            orows.append(_DOT(A2[bi * _T:(bi + 1) * _T, :], Vp))
        O = jnp.concatenate(orows, axis=0)
        hT = _ln(hT + _DOT(O, wo[:]), g1[:], b1[:])
        f = _DOT(jnp.maximum(_DOT(hT, f1[:]) + f1b[:], 0.0), f2[:]) + f2b[:]
        hT = _ln(hT + f, g2[:], b2[:])

    valid = jnp.where(pad, 0.0, 1.0).astype(jnp.float32)
    pooled_rows = []
    for bi in range(_B):
        vb = valid[bi:bi + 1, :]
        pooled_rows.append(_DOT(vb, hT[bi * _T:(bi + 1) * _T, :]))
    pooled = jnp.concatenate(pooled_rows, axis=0) / nf.astype(jnp.float32)

    z = jnp.maximum(_ln(_DOT(pooled, cw0_ref[:]) + cb0_ref[:],
                        cg0_ref[:], cbt0_ref[:]), 0.0)
    z = jnp.maximum(_ln(_DOT(z, cw1_ref[:]) + cb1_ref[:],
                        cg1_ref[:], cbt1_ref[:]), 0.0)
    out_ref[:] = _DOT(z, cw2_ref[:]) + cb2_ref[:]


def kernel(x_temporal, num_frames, gcn_W0, gcn_b0, gcn_W1, gcn_b1, gcn_W2,
           gcn_b2, tin_W, tin_b, Wq0, Wk0, Wv0, Wo0, ff1_0, ff1b_0, ff2_0,
           ff2b_0, ln1g_0, ln1b_0, ln2g_0, ln2b_0, Wq1, Wk1, Wv1, Wo1, ff1_1,
           ff1b_1, ff2_1, ff2b_1, ln1g_1, ln1b_1, ln2g_1, ln2b_1, cls_W0,
           cls_b0, cls_ln0g, cls_ln0b, cls_W1, cls_b1, cls_ln1g, cls_ln1b,
           cls_W2, cls_b2):
    Bb, Tt, Nn, Ff = x_temporal.shape
    xf = x_temporal.reshape(Bb * Tt * Nn, Ff)
    A = jnp.asarray(_band_adjacency())
    r2 = lambda a: a.reshape(1, -1)

    n_chunks = _G // _GB
    ge = pl.pallas_call(
        _spatial_kernel,
        grid=(n_chunks,),
        in_specs=[
            pl.BlockSpec((_ROWS, _F), lambda i: (i, 0)),
            pl.BlockSpec((_F, _H), lambda i: (0, 0)),
            pl.BlockSpec((1, _H), lambda i: (0, 0)),
            pl.BlockSpec((_H, _H), lambda i: (0, 0)),
            pl.BlockSpec((1, _H), lambda i: (0, 0)),
            pl.BlockSpec((_H, _H), lambda i: (0, 0)),
            pl.BlockSpec((1, _H), lambda i: (0, 0)),
            pl.BlockSpec((_N, _N), lambda i: (0, 0)),
        ],
        out_specs=pl.BlockSpec((_GB, 2 * _H), lambda i: (i, 0)),
        out_shape=jax.ShapeDtypeStruct((_G, 2 * _H), jnp.float32),
        compiler_params=pltpu.CompilerParams(
            dimension_semantics=("arbitrary",)),
    )(xf, gcn_W0, r2(gcn_b0), gcn_W1, r2(gcn_b1), gcn_W2, r2(gcn_b2), A)

    out = pl.pallas_call(
        _temporal_kernel,
        out_shape=jax.ShapeDtypeStruct((_B, 8), jnp.float32),
        scratch_shapes=[pltpu.VMEM((_B * _T, _HT), jnp.float32)],
    )(ge, num_frames, tin_W, r2(tin_b),
      Wq0, Wk0, Wv0, Wo0, ff1_0, r2(ff1b_0), ff2_0, r2(ff2b_0),
      r2(ln1g_0), r2(ln1b_0), r2(ln2g_0), r2(ln2b_0),
      Wq1, Wk1, Wv1, Wo1, ff1_1, r2(ff1b_1), ff2_1, r2(ff2b_1),
      r2(ln1g_1), r2(ln1b_1), r2(ln2g_1), r2(ln2b_1),
      cls_W0, r2(cls_b0), r2(cls_ln0g), r2(cls_ln0b),
      cls_W1, r2(cls_b1), r2(cls_ln1g), r2(cls_ln1b),
      cls_W2, r2(cls_b2))
    return out
